# baseline (speedup 1.0000x reference)
import functools

import jax
import jax.numpy as jnp
from jax import lax
from jax.experimental import pallas as pl
from jax.experimental.pallas import tpu as pltpu
from jax.experimental.pallas import tpu_sc as plsc

M_ROWS = 250000
D = 128
N = M_ROWS * D
NC = 2
NS = 16
NW = NC * NS
CHUNK = 12800
NCHUNK_G = N // CHUNK
GROUPS = CHUNK // 16
WGROUPS = CHUNK // 64
MAXCH = 79
SRCW = CHUNK + 8
SPAD = SRCW + 16
CROW = 80
NSLOT = 3

_mesh = plsc.VectorSubcoreMesh(core_axis_name="c", subcore_axis_name="s")
_params = pltpu.CompilerParams(needs_layout_passes=False)


def _wid():
    return lax.axis_index("s") * NC + lax.axis_index("c")


def _assign(w):
    start = 78 * w + jnp.minimum(w, 4)
    n = jnp.where(w < 4, 79, 78)
    return start, n


def _bytesums(v):
    u = v + (v << 8)
    return u + (u << 16)


@functools.partial(
    pl.kernel,
    out_type=jax.ShapeDtypeStruct((NW * CROW,), jnp.int32),
    mesh=_mesh,
    compiler_params=_params,
    scratch_types=[
        pltpu.VMEM((CHUNK,), jnp.float32),
        pltpu.VMEM((CHUNK,), jnp.float32),
        pltpu.VMEM((CROW,), jnp.int32),
        pltpu.SemaphoreType.DMA,
        pltpu.SemaphoreType.DMA,
    ],
)
def _count_kernel(mk_hbm, counts_hbm, mb0, mb1, cbuf, sm0, sm1):
    w = _wid()
    start_c, n_w = _assign(w)
    mbufs = (mb0, mb1)
    sems = (sm0, sm1)
    zeros = jnp.zeros((16,), jnp.int32)
    lane = lax.iota(jnp.int32, 16)
    m0 = lane == 0
    for j in range(CROW // 16):
        cbuf[pl.ds(j * 16, 16)] = zeros

    def mask_copy(i, u):
        return pltpu.make_async_copy(
            mk_hbm.at[pl.ds((start_c + i) * CHUNK, CHUNK)], mbufs[u], sems[u])

    mask_copy(0, 0).start()

    def pair_body(p, _):
        for u in range(2):
            i = 2 * p + u

            @pl.when(i < n_w)
            def _():
                mask_copy(i, u).wait()

                @pl.when(i + 1 < n_w)
                def _():
                    mask_copy(i + 1, 1 - u).start()

                @plsc.parallel_loop(0, WGROUPS, unroll=4,
                                    carry=jnp.zeros((16,), jnp.float32))
                def acc(g, a):
                    m = mbufs[u]
                    v0 = m[pl.ds(g * 64, 16)]
                    v1 = m[pl.ds(g * 64 + 16, 16)]
                    v2 = m[pl.ds(g * 64 + 32, 16)]
                    v3 = m[pl.ds(g * 64 + 48, 16)]
                    return a + ((v0 + v1) + (v2 + v3))

                cnt = jnp.sum(acc).astype(jnp.int32)
                plsc.store_scatter(cbuf, [zeros + i], zeros + cnt, mask=m0)

        return 0

    lax.fori_loop(0, (MAXCH + 1) // 2, pair_body, 0)
    pltpu.sync_copy(cbuf, counts_hbm.at[pl.ds(w * CROW, CROW)])


@functools.partial(
    pl.kernel,
    out_type=jax.ShapeDtypeStruct((N,), jnp.float32),
    mesh=_mesh,
    compiler_params=_params,
    scratch_types=(
        [pltpu.VMEM((CHUNK,), jnp.float32) for _ in range(NSLOT)]
        + [pltpu.VMEM((CHUNK,), jnp.float32) for _ in range(NSLOT)]
        + [pltpu.VMEM((SPAD,), jnp.float32) for _ in range(NSLOT)]
        + [
            pltpu.VMEM((NW * CROW,), jnp.int32),
            pltpu.VMEM((96,), jnp.int32),
        ]
        + [pltpu.SemaphoreType.DMA for _ in range(3 * NSLOT)]
    ),
)
def _scatter_kernel(mf_hbm, x_hbm, src_hbm, counts_hbm, out_hbm,
                    mb0, mb1, mb2, ob0, ob1, ob2, sb0, sb1, sb2,
                    cbuf, bbuf,
                    si0, si1, si2, sx0, sx1, sx2, so0, so1, so2):
    mbufs = (mb0, mb1, mb2)
    obufs = (ob0, ob1, ob2)
    sbufs = (sb0, sb1, sb2)
    sem_in = (si0, si1, si2)
    sem_x = (sx0, sx1, sx2)
    sem_out = (so0, so1, so2)
    w = _wid()
    start_c, n_w = _assign(w)
    zeros = jnp.zeros((16,), jnp.int32)
    lane = lax.iota(jnp.int32, 16)

    pltpu.sync_copy(counts_hbm, cbuf)

    def sb_body(j, acc):
        return acc + cbuf[pl.ds(j * 16, 16)]

    shard_base = jnp.sum(
        lax.fori_loop(0, w * (CROW // 16), sb_body, zeros))

    def bb_body(j, carry_s):
        cvec = cbuf[pl.ds((w * (CROW // 16) + j) * 16, 16)]
        bbuf[pl.ds(j * 16, 16)] = plsc.cumsum(cvec) - cvec + carry_s
        return carry_s + jnp.sum(cvec)

    lax.fori_loop(0, CROW // 16, bb_body, shard_base)

    def chunk_base(i):
        return bbuf[pl.ds(i, 16)][0]

    def win_base(i):
        return pl.multiple_of(
            jnp.minimum(chunk_base(i) & -8, N - SRCW), 8)

    def in_copies(i, u):
        start = (start_c + i) * CHUNK
        m = pltpu.make_async_copy(
            mf_hbm.at[pl.ds(start, CHUNK)], mbufs[u], sem_in[u])
        s = pltpu.make_async_copy(
            src_hbm.at[pl.ds(win_base(i), SRCW)],
            sbufs[u].at[pl.ds(0, SRCW)], sem_in[u])
        xc = pltpu.make_async_copy(
            x_hbm.at[pl.ds(start, CHUNK)], obufs[u], sem_x[u])
        return m, s, xc

    def out_copy(i, u):
        start = (start_c + i) * CHUNK
        return pltpu.make_async_copy(
            obufs[u], out_hbm.at[pl.ds(start, CHUNK)], sem_out[u])

    def do_chunk(i, u):
        mbuf, obuf, sbuf = mbufs[u], obufs[u], sbufs[u]
        for c in in_copies(i, u):
            c.wait()

        nu = (u + 2) % NSLOT

        @pl.when(i + 2 < n_w)
        def _():
            @pl.when(i >= 1)
            def _():
                out_copy(i - 1, nu).wait()

            for c in in_copies(i + 2, nu):
                c.start()

        base_s = chunk_base(i)
        off0 = zeros + (base_s - win_base(i))

        @plsc.parallel_loop(0, GROUPS, unroll=8, carry=off0)
        def _(g, off):
            mk = mbuf[pl.ds(g * 16, 16)] != 0.0
            vals = plsc.load_expanded(sbuf.at[pl.ds(off[0], 16)], mask=mk)
            plsc.store_scatter(obuf, [lane + g * 16], vals, mask=mk)
            return off + plsc.all_reduce_population_count(mk)

        out_copy(i, u).start()

    for c in in_copies(0, 0) + in_copies(1, 1):
        c.start()

    def triple_body(p, _):
        for u in range(NSLOT):
            i = 3 * p + u

            @pl.when(i < n_w)
            def _():
                do_chunk(i, u)

        return 0

    lax.fori_loop(0, (MAXCH + NSLOT - 1) // NSLOT, triple_body, 0)
    for u in range(NSLOT):
        out_copy(0, u).wait()


def kernel(x, mask, src):
    xf = x.reshape(-1)
    sf = src.reshape(-1)
    mf = mask.astype(jnp.float32).reshape(-1)
    counts = _count_kernel(mf)
    outf = _scatter_kernel(mf, xf, sf, counts)
    return outf.reshape(x.shape)


# speedup vs baseline: 1.2679x; 1.2679x over previous
import functools

import jax
import jax.numpy as jnp
from jax import lax
from jax.experimental import pallas as pl
from jax.experimental.pallas import tpu as pltpu
from jax.experimental.pallas import tpu_sc as plsc

M_ROWS = 250000
D = 128
N = M_ROWS * D
NC = 2
NS = 16
NW = NC * NS
CHUNK = 12800
NCHUNK_G = N // CHUNK
GROUPS = CHUNK // 16
WGROUPS = CHUNK // 64
MAXCH = 79
SRCW = CHUNK + 8
SPAD = SRCW + 16
CROW = 80
NSLOT = 3

_mesh = plsc.VectorSubcoreMesh(core_axis_name="c", subcore_axis_name="s")
_params = pltpu.CompilerParams(needs_layout_passes=False)


def _wid():
    return lax.axis_index("s") * NC + lax.axis_index("c")


def _assign(w):
    start = 78 * w + jnp.minimum(w, 4)
    n = jnp.where(w < 4, 79, 78)
    return start, n


def _bytesums(v):
    u = v + (v << 8)
    return u + (u << 16)


@functools.partial(
    pl.kernel,
    out_type=jax.ShapeDtypeStruct((NW * CROW,), jnp.int32),
    mesh=_mesh,
    compiler_params=_params,
    scratch_types=[
        pltpu.VMEM((CHUNK,), jnp.float32),
        pltpu.VMEM((CHUNK,), jnp.float32),
        pltpu.VMEM((CROW,), jnp.int32),
        pltpu.SemaphoreType.DMA,
        pltpu.SemaphoreType.DMA,
    ],
)
def _count_kernel(mk_hbm, counts_hbm, mb0, mb1, cbuf, sm0, sm1):
    w = _wid()
    start_c, n_w = _assign(w)
    mbufs = (mb0, mb1)
    sems = (sm0, sm1)
    zeros = jnp.zeros((16,), jnp.int32)
    lane = lax.iota(jnp.int32, 16)
    m0 = lane == 0
    for j in range(CROW // 16):
        cbuf[pl.ds(j * 16, 16)] = zeros

    def mask_copy(i, u):
        return pltpu.make_async_copy(
            mk_hbm.at[pl.ds((start_c + i) * CHUNK, CHUNK)], mbufs[u], sems[u])

    mask_copy(0, 0).start()

    def pair_body(p, _):
        for u in range(2):
            i = 2 * p + u

            @pl.when(i < n_w)
            def _():
                mask_copy(i, u).wait()

                @pl.when(i + 1 < n_w)
                def _():
                    mask_copy(i + 1, 1 - u).start()

                @plsc.parallel_loop(0, WGROUPS, unroll=4,
                                    carry=jnp.zeros((16,), jnp.float32))
                def acc(g, a):
                    m = mbufs[u]
                    v0 = m[pl.ds(g * 64, 16)]
                    v1 = m[pl.ds(g * 64 + 16, 16)]
                    v2 = m[pl.ds(g * 64 + 32, 16)]
                    v3 = m[pl.ds(g * 64 + 48, 16)]
                    return a + ((v0 + v1) + (v2 + v3))

                cnt = jnp.sum(acc).astype(jnp.int32)
                plsc.store_scatter(cbuf, [zeros + i], zeros + cnt, mask=m0)

        return 0

    lax.fori_loop(0, (MAXCH + 1) // 2, pair_body, 0)
    pltpu.sync_copy(cbuf, counts_hbm.at[pl.ds(w * CROW, CROW)])


@functools.partial(
    pl.kernel,
    out_type=jax.ShapeDtypeStruct((N,), jnp.float32),
    mesh=_mesh,
    compiler_params=_params,
    scratch_types=(
        [pltpu.VMEM((CHUNK,), jnp.float32) for _ in range(NSLOT)]
        + [pltpu.VMEM((CHUNK,), jnp.float32) for _ in range(NSLOT)]
        + [pltpu.VMEM((SPAD,), jnp.float32) for _ in range(NSLOT)]
        + [
            pltpu.VMEM((NW * CROW,), jnp.int32),
            pltpu.VMEM((96,), jnp.int32),
        ]
        + [pltpu.SemaphoreType.DMA for _ in range(3 * NSLOT)]
    ),
)
def _scatter_kernel(mf_hbm, x_hbm, src_hbm, counts_hbm, out_hbm,
                    mb0, mb1, mb2, ob0, ob1, ob2, sb0, sb1, sb2,
                    cbuf, bbuf,
                    si0, si1, si2, sx0, sx1, sx2, so0, so1, so2):
    mbufs = (mb0, mb1, mb2)
    obufs = (ob0, ob1, ob2)
    sbufs = (sb0, sb1, sb2)
    sem_in = (si0, si1, si2)
    sem_x = (sx0, sx1, sx2)
    sem_out = (so0, so1, so2)
    w = _wid()
    start_c, n_w = _assign(w)
    zeros = jnp.zeros((16,), jnp.int32)
    lane = lax.iota(jnp.int32, 16)

    pltpu.sync_copy(counts_hbm, cbuf)

    def sb_body(j, acc):
        return acc + cbuf[pl.ds(j * 16, 16)]

    shard_base = jnp.sum(
        lax.fori_loop(0, w * (CROW // 16), sb_body, zeros))

    def bb_body(j, carry_s):
        cvec = cbuf[pl.ds((w * (CROW // 16) + j) * 16, 16)]
        bbuf[pl.ds(j * 16, 16)] = plsc.cumsum(cvec) - cvec + carry_s
        return carry_s + jnp.sum(cvec)

    lax.fori_loop(0, CROW // 16, bb_body, shard_base)

    def chunk_base(i):
        return bbuf[pl.ds(i, 16)][0]

    def win_base(i):
        return pl.multiple_of(
            jnp.minimum(chunk_base(i) & -8, N - SRCW), 8)

    def in_copies(i, u):
        start = (start_c + i) * CHUNK
        m = pltpu.make_async_copy(
            mf_hbm.at[pl.ds(start, CHUNK)], mbufs[u], sem_in[u])
        s = pltpu.make_async_copy(
            src_hbm.at[pl.ds(win_base(i), SRCW)],
            sbufs[u].at[pl.ds(0, SRCW)], sem_in[u])
        xc = pltpu.make_async_copy(
            x_hbm.at[pl.ds(start, CHUNK)], obufs[u], sem_x[u])
        return m, s, xc

    def out_copy(i, u):
        start = (start_c + i) * CHUNK
        return pltpu.make_async_copy(
            obufs[u], out_hbm.at[pl.ds(start, CHUNK)], sem_out[u])

    def do_chunk(i, u):
        mbuf, obuf, sbuf = mbufs[u], obufs[u], sbufs[u]
        for c in in_copies(i, u):
            c.wait()

        nu = (u + 2) % NSLOT

        @pl.when(i + 2 < n_w)
        def _():
            @pl.when(i >= 1)
            def _():
                out_copy(i - 1, nu).wait()

            for c in in_copies(i + 2, nu):
                c.start()

        base_s = chunk_base(i)
        off0 = zeros + (base_s - win_base(i))

        @plsc.parallel_loop(0, GROUPS, unroll=4, carry=off0)
        def _(g, off):
            mk = mbuf[pl.ds(g * 16, 16)] != 0.0
            vals = plsc.load_expanded(sbuf.at[pl.ds(off[0], 16)], mask=mk)
            plsc.store_scatter(obuf, [lane + g * 16], vals, mask=mk)
            return off + plsc.all_reduce_population_count(mk)

        out_copy(i, u).start()

    for c in in_copies(0, 0) + in_copies(1, 1):
        c.start()

    def triple_body(p, _):
        for u in range(NSLOT):
            i = 3 * p + u

            @pl.when(i < n_w)
            def _():
                do_chunk(i, u)

        return 0

    lax.fori_loop(0, (MAXCH + NSLOT - 1) // NSLOT, triple_body, 0)
    for u in range(NSLOT):
        out_copy(0, u).wait()


def kernel(x, mask, src):
    xf = x.reshape(-1)
    sf = src.reshape(-1)
    mf = mask.astype(jnp.float32).reshape(-1)
    counts = _count_kernel(mf)
    outf = _scatter_kernel(mf, xf, sf, counts)
    return outf.reshape(x.shape)


# speedup vs baseline: 1.2701x; 1.0017x over previous
import functools

import jax
import jax.numpy as jnp
from jax import lax
from jax.experimental import pallas as pl
from jax.experimental.pallas import tpu as pltpu
from jax.experimental.pallas import tpu_sc as plsc

M_ROWS = 250000
D = 128
N = M_ROWS * D
NC = 2
NS = 16
NW = NC * NS
CHUNK = 12800
NCHUNK_G = N // CHUNK
GROUPS = CHUNK // 16
WGROUPS = CHUNK // 64
MAXCH = 79
SRCW = CHUNK + 8
SPAD = SRCW + 16
CROW = 80
NSLOT = 3

_mesh = plsc.VectorSubcoreMesh(core_axis_name="c", subcore_axis_name="s")
_params = pltpu.CompilerParams(needs_layout_passes=False)


def _wid():
    return lax.axis_index("s") * NC + lax.axis_index("c")


def _assign(w):
    start = 78 * w + jnp.minimum(w, 4)
    n = jnp.where(w < 4, 79, 78)
    return start, n


def _bytesums(v):
    u = v + (v << 8)
    return u + (u << 16)


@functools.partial(
    pl.kernel,
    out_type=jax.ShapeDtypeStruct((NW * CROW,), jnp.int32),
    mesh=_mesh,
    compiler_params=_params,
    scratch_types=[
        pltpu.VMEM((CHUNK,), jnp.float32),
        pltpu.VMEM((CHUNK,), jnp.float32),
        pltpu.VMEM((CROW,), jnp.int32),
        pltpu.SemaphoreType.DMA,
        pltpu.SemaphoreType.DMA,
    ],
)
def _count_kernel(mk_hbm, counts_hbm, mb0, mb1, cbuf, sm0, sm1):
    w = _wid()
    start_c, n_w = _assign(w)
    mbufs = (mb0, mb1)
    sems = (sm0, sm1)
    zeros = jnp.zeros((16,), jnp.int32)
    lane = lax.iota(jnp.int32, 16)
    m0 = lane == 0
    for j in range(CROW // 16):
        cbuf[pl.ds(j * 16, 16)] = zeros

    def mask_copy(i, u):
        return pltpu.make_async_copy(
            mk_hbm.at[pl.ds((start_c + i) * CHUNK, CHUNK)], mbufs[u], sems[u])

    mask_copy(0, 0).start()

    def pair_body(p, _):
        for u in range(2):
            i = 2 * p + u

            @pl.when(i < n_w)
            def _():
                mask_copy(i, u).wait()

                @pl.when(i + 1 < n_w)
                def _():
                    mask_copy(i + 1, 1 - u).start()

                @plsc.parallel_loop(0, WGROUPS, unroll=2,
                                    carry=jnp.zeros((16,), jnp.float32))
                def acc(g, a):
                    m = mbufs[u]
                    v0 = m[pl.ds(g * 64, 16)]
                    v1 = m[pl.ds(g * 64 + 16, 16)]
                    v2 = m[pl.ds(g * 64 + 32, 16)]
                    v3 = m[pl.ds(g * 64 + 48, 16)]
                    return a + ((v0 + v1) + (v2 + v3))

                cnt = jnp.sum(acc).astype(jnp.int32)
                plsc.store_scatter(cbuf, [zeros + i], zeros + cnt, mask=m0)

        return 0

    lax.fori_loop(0, (MAXCH + 1) // 2, pair_body, 0)
    pltpu.sync_copy(cbuf, counts_hbm.at[pl.ds(w * CROW, CROW)])


@functools.partial(
    pl.kernel,
    out_type=jax.ShapeDtypeStruct((N,), jnp.float32),
    mesh=_mesh,
    compiler_params=_params,
    scratch_types=(
        [pltpu.VMEM((CHUNK,), jnp.float32) for _ in range(NSLOT)]
        + [pltpu.VMEM((CHUNK,), jnp.float32) for _ in range(NSLOT)]
        + [pltpu.VMEM((SPAD,), jnp.float32) for _ in range(NSLOT)]
        + [
            pltpu.VMEM((NW * CROW,), jnp.int32),
            pltpu.VMEM((96,), jnp.int32),
        ]
        + [pltpu.SemaphoreType.DMA for _ in range(3 * NSLOT)]
    ),
)
def _scatter_kernel(mf_hbm, x_hbm, src_hbm, counts_hbm, out_hbm,
                    mb0, mb1, mb2, ob0, ob1, ob2, sb0, sb1, sb2,
                    cbuf, bbuf,
                    si0, si1, si2, sx0, sx1, sx2, so0, so1, so2):
    mbufs = (mb0, mb1, mb2)
    obufs = (ob0, ob1, ob2)
    sbufs = (sb0, sb1, sb2)
    sem_in = (si0, si1, si2)
    sem_x = (sx0, sx1, sx2)
    sem_out = (so0, so1, so2)
    w = _wid()
    start_c, n_w = _assign(w)
    zeros = jnp.zeros((16,), jnp.int32)
    lane = lax.iota(jnp.int32, 16)

    pltpu.sync_copy(counts_hbm, cbuf)

    def sb_body(j, acc):
        return acc + cbuf[pl.ds(j * 16, 16)]

    shard_base = jnp.sum(
        lax.fori_loop(0, w * (CROW // 16), sb_body, zeros))

    def bb_body(j, carry_s):
        cvec = cbuf[pl.ds((w * (CROW // 16) + j) * 16, 16)]
        bbuf[pl.ds(j * 16, 16)] = plsc.cumsum(cvec) - cvec + carry_s
        return carry_s + jnp.sum(cvec)

    lax.fori_loop(0, CROW // 16, bb_body, shard_base)

    def chunk_base(i):
        return bbuf[pl.ds(i, 16)][0]

    def win_base(i):
        return pl.multiple_of(
            jnp.minimum(chunk_base(i) & -8, N - SRCW), 8)

    def in_copies(i, u):
        start = (start_c + i) * CHUNK
        m = pltpu.make_async_copy(
            mf_hbm.at[pl.ds(start, CHUNK)], mbufs[u], sem_in[u])
        s = pltpu.make_async_copy(
            src_hbm.at[pl.ds(win_base(i), SRCW)],
            sbufs[u].at[pl.ds(0, SRCW)], sem_in[u])
        xc = pltpu.make_async_copy(
            x_hbm.at[pl.ds(start, CHUNK)], obufs[u], sem_x[u])
        return m, s, xc

    def out_copy(i, u):
        start = (start_c + i) * CHUNK
        return pltpu.make_async_copy(
            obufs[u], out_hbm.at[pl.ds(start, CHUNK)], sem_out[u])

    def do_chunk(i, u):
        mbuf, obuf, sbuf = mbufs[u], obufs[u], sbufs[u]
        for c in in_copies(i, u):
            c.wait()

        nu = (u + 2) % NSLOT

        @pl.when(i + 2 < n_w)
        def _():
            @pl.when(i >= 1)
            def _():
                out_copy(i - 1, nu).wait()

            for c in in_copies(i + 2, nu):
                c.start()

        base_s = chunk_base(i)
        off0 = zeros + (base_s - win_base(i))

        @plsc.parallel_loop(0, GROUPS, unroll=4, carry=off0)
        def _(g, off):
            mk = mbuf[pl.ds(g * 16, 16)] != 0.0
            vals = plsc.load_expanded(sbuf.at[pl.ds(off[0], 16)], mask=mk)
            plsc.store_scatter(obuf, [lane + g * 16], vals, mask=mk)
            return off + plsc.all_reduce_population_count(mk)

        out_copy(i, u).start()

    for c in in_copies(0, 0) + in_copies(1, 1):
        c.start()

    def triple_body(p, _):
        for u in range(NSLOT):
            i = 3 * p + u

            @pl.when(i < n_w)
            def _():
                do_chunk(i, u)

        return 0

    lax.fori_loop(0, (MAXCH + NSLOT - 1) // NSLOT, triple_body, 0)
    for u in range(NSLOT):
        out_copy(0, u).wait()


def kernel(x, mask, src):
    xf = x.reshape(-1)
    sf = src.reshape(-1)
    mf = mask.astype(jnp.float32).reshape(-1)
    counts = _count_kernel(mf)
    outf = _scatter_kernel(mf, xf, sf, counts)
    return outf.reshape(x.shape)
